# baseline (speedup 1.0000x reference)
import functools

import jax
import jax.numpy as jnp
from jax import lax
from jax.experimental import pallas as pl
from jax.experimental.pallas import tpu as pltpu
from jax.experimental.pallas import tpu_sc as plsc

_NC = 2
_NS = 16
_NW = _NC * _NS
_NBUF = 4
_K = 128


@functools.lru_cache(maxsize=None)
def _make_kernel(B, L, D):
    K = _K
    N = B * L
    G = N // K
    cpw = G // _NW
    sidx = (cpw + 15) // 8 * 8
    assert N % K == 0 and G % _NW == 0 and D % 16 == 0 and L % 8 == 0

    mesh = plsc.VectorSubcoreMesh(core_axis_name="c", subcore_axis_name="s")

    @functools.partial(
        pl.kernel,
        mesh=mesh,
        out_type=jax.ShapeDtypeStruct((N, D), jnp.float32),
        scratch_types=[
            pltpu.VMEM((sidx, K), jnp.int32),
            pltpu.VMEM((_NBUF, K, D), jnp.float32),
            pltpu.VMEM((2 * L, D), jnp.float32),
            [pltpu.SemaphoreType.DMA] * _NBUF,
            [pltpu.SemaphoreType.DMA] * _NBUF,
        ],
    )
    def k(idx_hbm, table_hbm, pos_hbm, out_hbm, idx_v, rows_v, pos_v,
          gsems, osems):
        wid = lax.axis_index("s") * _NC + lax.axis_index("c")
        q0 = wid * cpw
        base8 = jnp.minimum(q0 // 8 * 8, G - sidx)
        j0 = q0 - base8

        pltpu.sync_copy(pos_hbm, pos_v.at[pl.ds(0, L)])
        pltpu.sync_copy(pos_hbm, pos_v.at[pl.ds(L, L)])
        pltpu.sync_copy(idx_hbm.at[pl.ds(base8, sidx)], idx_v)

        def gather(q, b):
            return pltpu.make_async_copy(
                table_hbm.at[idx_v.at[j0 + q]], rows_v.at[b], gsems[b])

        def wback(q, b):
            return pltpu.make_async_copy(
                rows_v.at[b], out_hbm.at[pl.ds((q0 + q) * K, K)], osems[b])

        def step(q, b):
            gather(q, b).wait()
            s0 = lax.rem((q0 + q) * K, L)

            def tok_body(t, c2, b=b):
                for d in range(D // 16):
                    sl = pl.ds(d * 16, 16)
                    plsc.addupdate(rows_v.at[b, t, sl], pos_v[s0 + t, sl])
                return c2

            lax.fori_loop(0, K, tok_body, 0)
            wback(q, b).start()

            pb = (b - 1) % _NBUF

            @pl.when(q >= 1)
            def _():
                wback(q - 1, pb).wait()

            @pl.when(q + _NBUF - 1 < cpw)
            def _():
                gather(q + _NBUF - 1, pb).start()

        for b in range(_NBUF - 1):
            gather(b, b).start()

        n_main = cpw // _NBUF * _NBUF

        def super_body(i, carry):
            g = i * _NBUF
            for b in range(_NBUF):
                step(g + b, b)
            return carry

        lax.fori_loop(0, cpw // _NBUF, super_body, 0)
        for q in range(n_main, cpw):
            step(q, q % _NBUF)
        wback(cpw - 1, (cpw - 1) % _NBUF).wait()

    return k


def kernel(inputs, token_table, pos_table):
    B, L = inputs.shape
    _, D = token_table.shape
    k = _make_kernel(B, L, D)
    out = k(
        inputs.astype(jnp.int32).reshape(B * L // _K, _K),
        token_table,
        pos_table,
    )
    return out.reshape(B, L, D)


# speedup vs baseline: 2.4213x; 2.4213x over previous
import functools

import jax
import jax.numpy as jnp
from jax import lax
from jax.experimental import pallas as pl
from jax.experimental.pallas import tpu as pltpu
from jax.experimental.pallas import tpu_sc as plsc

_NC = 2
_NS = 16
_NW = _NC * _NS
_NBUF = 4
_KA = 88


@functools.lru_cache(maxsize=None)
def _make_kernel(B, L, D):
    KA = _KA
    KB = L - KA
    rpw = B // _NW
    cpw = 2 * rpw
    assert B % _NW == 0 and cpw % _NBUF == 0 and D % 16 == 0
    assert KA % 8 == 0 and KB % 8 == 0 and KA <= 128 and KB <= 128

    mesh = plsc.VectorSubcoreMesh(core_axis_name="c", subcore_axis_name="s")

    @functools.partial(
        pl.kernel,
        mesh=mesh,
        out_type=jax.ShapeDtypeStruct((B, L, D), jnp.float32),
        scratch_types=[
            pltpu.VMEM((rpw, KA), jnp.int32),
            pltpu.VMEM((rpw, KB), jnp.int32),
            pltpu.VMEM((2, KA, D), jnp.float32),
            pltpu.VMEM((2, KB, D), jnp.float32),
            pltpu.VMEM((L, D), jnp.float32),
            [pltpu.SemaphoreType.DMA] * _NBUF,
            [pltpu.SemaphoreType.DMA] * _NBUF,
        ],
    )
    def k(ia_hbm, ib_hbm, table_hbm, pos_hbm, out_hbm, idx_a, idx_b,
          rows_a, rows_b, pos_v, gsems, osems):
        wid = lax.axis_index("s") * _NC + lax.axis_index("c")
        row0 = wid * rpw

        pltpu.sync_copy(pos_hbm, pos_v)
        pltpu.sync_copy(ia_hbm.at[pl.ds(row0, rpw)], idx_a)
        pltpu.sync_copy(ib_hbm.at[pl.ds(row0, rpw)], idx_b)

        def gather(q, b):
            if b % 2 == 0:
                return pltpu.make_async_copy(
                    table_hbm.at[idx_a.at[q // 2]], rows_a.at[b // 2],
                    gsems[b])
            return pltpu.make_async_copy(
                table_hbm.at[idx_b.at[q // 2]], rows_b.at[b // 2], gsems[b])

        def wback(q, b):
            if b % 2 == 0:
                return pltpu.make_async_copy(
                    rows_a.at[b // 2],
                    out_hbm.at[row0 + q // 2, pl.ds(0, KA)], osems[b])
            return pltpu.make_async_copy(
                rows_b.at[b // 2],
                out_hbm.at[row0 + q // 2, pl.ds(KA, KB)], osems[b])

        for b in range(_NBUF - 1):
            gather(b, b).start()

        def super_body(i, carry):
            g = i * _NBUF
            for b in range(_NBUF):
                q = g + b
                gather(q, b).wait()

                rows_v = rows_a if b % 2 == 0 else rows_b
                n_tok = KA if b % 2 == 0 else KB
                off = 0 if b % 2 == 0 else KA

                def tok_body(t, c2, rows_v=rows_v, b=b, off=off):
                    for d in range(D // 16):
                        sl = pl.ds(d * 16, 16)
                        plsc.addupdate(rows_v.at[b // 2, t, sl],
                                       pos_v[off + t, sl])
                    return c2

                lax.fori_loop(0, n_tok, tok_body, 0)

                pb = (b - 1) % _NBUF

                @pl.when(q >= 1)
                def _(q=q, pb=pb):
                    wback(q - 1, pb).wait()

                @pl.when(q + _NBUF - 1 < cpw)
                def _(q=q, pb=pb):
                    gather(q + _NBUF - 1, pb).start()

                wback(q, b).start()

            return carry

        lax.fori_loop(0, cpw // _NBUF, super_body, 0)
        wback(cpw - 1, _NBUF - 1).wait()

    return k


def kernel(inputs, token_table, pos_table):
    B, L = inputs.shape
    _, D = token_table.shape
    k = _make_kernel(B, L, D)
    idx = inputs.astype(jnp.int32)
    return k(idx[:, :_KA], idx[:, _KA:], token_table, pos_table)
